# Initial kernel scaffold; baseline (speedup 1.0000x reference)
import jax
import jax.numpy as jnp
from jax.experimental import pallas as pl


def kernel(x, gate_w, gate_b, expert_w, expert_b):
    raise NotImplementedError("write your pallas kernel here")


# speedup vs baseline: 3.3315x; 3.3315x over previous
import functools

import jax
import jax.numpy as jnp
from jax.experimental import pallas as pl
from jax.experimental.pallas import tpu as pltpu


def _moe_block_kernel(x_ref, gw_ref, gb_ref, ew_ref, eb_ref, out_ref):
    x = x_ref[:]
    logits = (
        jnp.dot(x, gw_ref[:], preferred_element_type=jnp.float32) + gb_ref[:]
    )
    g = jax.nn.softmax(logits, axis=-1)
    acc = jnp.dot(g, eb_ref[:], preferred_element_type=jnp.float32)
    num_experts = ew_ref.shape[0]
    for e in range(num_experts):
        y = jnp.dot(x, ew_ref[e], preferred_element_type=jnp.float32)
        acc = acc + g[:, e : e + 1] * y
    out_ref[:] = acc


@jax.jit
def kernel(x, gate_w, gate_b, expert_w, expert_b):
    n, d = x.shape
    e = expert_w.shape[0]
    f = expert_w.shape[2]
    tile = 1024
    grid = (n // tile,)
    gate_b2 = gate_b.reshape(1, e)
    return pl.pallas_call(
        _moe_block_kernel,
        grid=grid,
        in_specs=[
            pl.BlockSpec((tile, d), lambda i: (i, 0)),
            pl.BlockSpec((d, e), lambda i: (0, 0)),
            pl.BlockSpec((1, e), lambda i: (0, 0)),
            pl.BlockSpec((e, d, f), lambda i: (0, 0, 0)),
            pl.BlockSpec((e, f), lambda i: (0, 0)),
        ],
        out_specs=pl.BlockSpec((tile, f), lambda i: (i, 0)),
        out_shape=jax.ShapeDtypeStruct((n, f), jnp.float32),
        compiler_params=pltpu.CompilerParams(
            dimension_semantics=("parallel",),
        ),
    )(x, gate_w, gate_b2, expert_w, expert_b)
